# baseline (speedup 1.0000x reference)
import functools

import jax
import jax.numpy as jnp
from jax import lax
from jax.experimental import pallas as pl
from jax.experimental.pallas import tpu as pltpu
from jax.experimental.pallas import tpu_sc as plsc

_V = 100000
_D = 2048
_B = 4096

_info = plsc.get_sparse_core_info()
_NW = _info.num_cores * _info.num_subcores
_BPW = _B // _NW
_C = 16
_NCH = _BPW // _C
_NBUF = 3

_mesh = plsc.VectorSubcoreMesh(core_axis_name="c", subcore_axis_name="s")


@functools.partial(
    pl.kernel,
    mesh=_mesh,
    out_type=jax.ShapeDtypeStruct((_B, _D), jnp.float32),
    scratch_types=[
        pltpu.VMEM((_NCH, _C), jnp.int32),
        pltpu.VMEM((_NBUF, _C, _D), jnp.float32),
        pltpu.SemaphoreType.DMA((_NBUF,)),
        pltpu.SemaphoreType.DMA((_NBUF,)),
    ],
)
def _gather_kernel(pids_hbm, ctx_hbm, out_hbm, idx_v, bufs, gsems, wsems):
    wid = lax.axis_index("s") * _info.num_cores + lax.axis_index("c")
    base = wid * _BPW

    pltpu.sync_copy(pids_hbm.at[pl.ds(wid * _NCH, _NCH)], idx_v)

    gathers = [None] * _NCH
    writes = [None] * _NCH

    def start_gather(c):
        gathers[c] = pltpu.async_copy(
            ctx_hbm.at[idx_v.at[c]], bufs.at[c % _NBUF], gsems.at[c % _NBUF]
        )

    start_gather(0)
    if _NCH > 1:
        start_gather(1)
    for c in range(_NCH):
        if c + 2 < _NCH:
            if c >= 1:
                writes[c - 1].wait()
            start_gather(c + 2)
        gathers[c].wait()
        writes[c] = pltpu.async_copy(
            bufs.at[c % _NBUF],
            out_hbm.at[pl.ds(base + c * _C, _C)],
            wsems.at[c % _NBUF],
        )
    for c in range(max(0, _NCH - 2), _NCH):
        writes[c].wait()


def kernel(pids, ctx):
    pids32 = pids.astype(jnp.int32).reshape(_B // _C, _C)
    table = ctx.reshape(_V, _D)
    out = _gather_kernel(pids32, table)
    return out.reshape(_B, 4, 512)


# speedup vs baseline: 15.4198x; 15.4198x over previous
import functools

import jax
import jax.numpy as jnp
from jax import lax
from jax.experimental import pallas as pl
from jax.experimental.pallas import tpu as pltpu
from jax.experimental.pallas import tpu_sc as plsc

_V = 100000
_N = 4
_E = 512
_B = 4096

_info = plsc.get_sparse_core_info()
_NW = _info.num_cores * _info.num_subcores
_BPW = _B // _NW
_C = 16
_NCH = _BPW // _C
_NBUF = 3

_mesh = plsc.VectorSubcoreMesh(core_axis_name="c", subcore_axis_name="s")


@functools.partial(
    pl.kernel,
    mesh=_mesh,
    out_type=jax.ShapeDtypeStruct((_B, _N, _E), jnp.float32),
    scratch_types=[
        pltpu.VMEM((_BPW,), jnp.int32),
        pltpu.VMEM((_NBUF, _C, _N, _E), jnp.float32),
        pltpu.SemaphoreType.DMA((_NBUF,)),
        pltpu.SemaphoreType.DMA((_NBUF,)),
    ],
)
def _gather_kernel(pids_hbm, ctx_hbm, out_hbm, idx_v, bufs, gsems, wsems):
    wid = lax.axis_index("s") * _info.num_cores + lax.axis_index("c")
    base = wid * _BPW

    pltpu.sync_copy(pids_hbm.at[pl.ds(base, _BPW)], idx_v)

    gathers = [None] * _NCH
    writes = [None] * _NCH

    def start_gather(c):
        gathers[c] = pltpu.async_copy(
            ctx_hbm.at[idx_v.at[pl.ds(c * _C, _C)]],
            bufs.at[c % _NBUF],
            gsems.at[c % _NBUF],
        )

    start_gather(0)
    if _NCH > 1:
        start_gather(1)
    for c in range(_NCH):
        if c + 2 < _NCH:
            if c >= 1:
                writes[c - 1].wait()
            start_gather(c + 2)
        gathers[c].wait()
        writes[c] = pltpu.async_copy(
            bufs.at[c % _NBUF],
            out_hbm.at[pl.ds(base + c * _C, _C)],
            wsems.at[c % _NBUF],
        )
    for c in range(max(0, _NCH - 2), _NCH):
        writes[c].wait()


def kernel(pids, ctx):
    return _gather_kernel(pids.astype(jnp.int32), ctx)


# speedup vs baseline: 15.8986x; 1.0311x over previous
import functools

import jax
import jax.numpy as jnp
from jax import lax
from jax.experimental import pallas as pl
from jax.experimental.pallas import tpu as pltpu
from jax.experimental.pallas import tpu_sc as plsc

_V = 100000
_N = 4
_E = 512
_B = 4096

_info = plsc.get_sparse_core_info()
_NW = _info.num_cores * _info.num_subcores
_BPW = _B // _NW
_C = 8
_NCH = _BPW // _C
_NBUF = 6

_mesh = plsc.VectorSubcoreMesh(core_axis_name="c", subcore_axis_name="s")


@functools.partial(
    pl.kernel,
    mesh=_mesh,
    out_type=jax.ShapeDtypeStruct((_B, _N, _E), jnp.float32),
    scratch_types=[
        pltpu.VMEM((_BPW,), jnp.int32),
        pltpu.VMEM((_NBUF, _C, _N, _E), jnp.float32),
        pltpu.SemaphoreType.DMA((_NBUF,)),
        pltpu.SemaphoreType.DMA((_NBUF,)),
    ],
)
def _gather_kernel(pids_hbm, ctx_hbm, out_hbm, idx_v, bufs, gsems, wsems):
    wid = lax.axis_index("s") * _info.num_cores + lax.axis_index("c")
    base = wid * _BPW

    pltpu.sync_copy(pids_hbm.at[pl.ds(base, _BPW)], idx_v)

    gathers = [None] * _NCH
    writes = [None] * _NCH

    def start_gather(c):
        gathers[c] = pltpu.async_copy(
            ctx_hbm.at[idx_v.at[pl.ds(c * _C, _C)]],
            bufs.at[c % _NBUF],
            gsems.at[c % _NBUF],
        )

    for c in range(min(_NBUF - 1, _NCH)):
        start_gather(c)
    for c in range(_NCH):
        if c + _NBUF - 1 < _NCH:
            if c >= 1:
                writes[c - 1].wait()
            start_gather(c + _NBUF - 1)
        gathers[c].wait()
        writes[c] = pltpu.async_copy(
            bufs.at[c % _NBUF],
            out_hbm.at[pl.ds(base + c * _C, _C)],
            wsems.at[c % _NBUF],
        )
    for c in range(max(0, _NCH - _NBUF + 1), _NCH):
        writes[c].wait()


def kernel(pids, ctx):
    return _gather_kernel(pids.astype(jnp.int32), ctx)
